# baseline (speedup 1.0000x reference)
import jax
import jax.numpy as jnp
from jax import lax
from jax.experimental import pallas as pl
from jax.experimental.pallas import tpu as pltpu
from jax.experimental.pallas import tpu_sc as plsc

N = 10000
E = 160000
D = 256
H = 256
O = 256

NC = 2
NS = 16
L = 16
NW = NC * NS

CH = 9984
CH_LAST = E - 15 * CH
HH = CH // 2
HH_LAST = CH_LAST // 2
MIN_VREGS = CH_LAST // L
H_ITERS = HH_LAST // L
U = 16
ZU = 5
MU = 16

_INT32_MAX = 2147483647


def _sc_body(ei_hbm, deg_out, cnt_out, idx_out,
             ei_v, deg_v, cnt_v, stage_v, shared_mins, allmin_v, dma_sem,
             out_sem):
    c_idx = lax.axis_index("c")
    s_idx = lax.axis_index("s")
    wid = s_idx * NC + c_idx

    iota = lax.iota(jnp.int32, L)
    ones = jnp.full((L,), 1.0, jnp.float32)
    zeros = jnp.zeros((L,), jnp.float32)

    ei_dma = pltpu.async_copy(ei_hbm.at[:, pl.ds(s_idx * CH, CH_LAST)],
                              ei_v, dma_sem)

    def zero_body(i, carry):
        base = i * (L * ZU)
        for u in range(ZU):
            off = base + u * L
            deg_v[pl.ds(off, L)] = zeros
            cnt_v[pl.ds(off, L)] = zeros
        return carry
    lax.fori_loop(0, (N // L) // ZU, zero_body, 0)

    ei_dma.wait()

    def min_body(i, acc):
        base = i * (L * MU)
        for u in range(MU):
            off = base + u * L
            acc = jnp.minimum(acc, ei_v[0, pl.ds(off, L)])
        return acc
    acc = lax.fori_loop(0, MIN_VREGS // MU, min_body,
                        jnp.full((L,), _INT32_MAX, jnp.int32))

    hlen = jnp.where(s_idx == NS - 1, HH_LAST, HH)
    hbase = c_idx * hlen

    def deg_body(i, carry):
        base = i * (L * U)
        for u in range(U):
            off = base + u * L
            cvals = ei_v[1, pl.ds(hbase + off, L)]
            plsc.addupdate_scatter(deg_v, [cvals], ones,
                                   mask=(off + iota) < hlen)
        return carry
    lax.fori_loop(0, H_ITERS // U, deg_body, 0)

    deg_dma = pltpu.async_copy(deg_v, deg_out.at[wid], out_sem)

    stage_v[...] = acc
    pltpu.sync_copy(stage_v, shared_mins.at[pl.ds(s_idx * L, L)])
    plsc.subcore_barrier()
    pltpu.sync_copy(shared_mins, allmin_v)
    macc = allmin_v[pl.ds(0, L)]
    for t in range(1, NS):
        macc = jnp.minimum(macc, allmin_v[pl.ds(t * L, L)])
    gmin = jnp.min(macc)

    def cnt_body(i, carry):
        base = i * (L * U)
        for u in range(U):
            off = base + u * L
            cvals = ei_v[1, pl.ds(hbase + off, L)]
            rvals = ei_v[0, pl.ds(hbase + off, L)]
            plsc.addupdate_scatter(
                cnt_v, [rvals], ones,
                mask=((off + iota) < hlen) & (cvals == gmin))
        return carry
    lax.fori_loop(0, H_ITERS // U, cnt_body, 0)

    pltpu.sync_copy(cnt_v, cnt_out.at[wid])
    deg_dma.wait()

    @pl.when(wid == 0)
    def _():
        stage_v[...] = jnp.full((L,), 0, jnp.int32) + gmin
        pltpu.sync_copy(stage_v, idx_out)


@jax.jit
def _sc_hist(edge_index):
    kern = pl.kernel(
        _sc_body,
        out_type=(
            jax.ShapeDtypeStruct((NW, N), jnp.float32),
            jax.ShapeDtypeStruct((NW, N), jnp.float32),
            jax.ShapeDtypeStruct((L,), jnp.int32),
        ),
        mesh=plsc.VectorSubcoreMesh(core_axis_name="c", subcore_axis_name="s"),
        compiler_params=pltpu.CompilerParams(needs_layout_passes=False),
        scratch_types=[
            pltpu.VMEM((2, CH_LAST), jnp.int32),
            pltpu.VMEM((N,), jnp.float32),
            pltpu.VMEM((N,), jnp.float32),
            pltpu.VMEM((L,), jnp.int32),
            pltpu.VMEM_SHARED((NS * L,), jnp.int32),
            pltpu.VMEM((NS * L,), jnp.int32),
            pltpu.SemaphoreType.DMA,
            pltpu.SemaphoreType.DMA,
        ],
    )
    return kern(edge_index)


def _tc_body(idx_ref, degp_ref, cntp_ref, x_ref, w1_ref, wfc_ref,
             bias_ref, bfc_ref, y_ref):
    deg = jnp.sum(degp_ref[...], axis=0, keepdims=True)
    cnt = jnp.sum(cntp_ref[...], axis=0, keepdims=True)
    w = cnt * jnp.sqrt(deg)
    idx = idx_ref[0]
    onehot = (lax.broadcasted_iota(jnp.int32, (1, N), 1) == idx)
    scale = jnp.sqrt(jnp.sum(jnp.where(onehot, deg, 0.0)))
    s = lax.dot_general(w, x_ref[...], (((1,), (0,)), ((), ())),
                        preferred_element_type=jnp.float32,
                        precision=lax.Precision.HIGHEST)
    z = lax.dot_general(s, w1_ref[...], (((1,), (1,)), ((), ())),
                        preferred_element_type=jnp.float32,
                        precision=lax.Precision.HIGHEST)
    out_row = scale * z + bias_ref[...]
    y = lax.dot_general(out_row, wfc_ref[...], (((1,), (1,)), ((), ())),
                        preferred_element_type=jnp.float32,
                        precision=lax.Precision.HIGHEST) + bfc_ref[...]
    y_ref[...] = y


@jax.jit
def _tc_finish(idxv, degp, cntp, x, W1, Wfc, bias2, bfc2):
    return pl.pallas_call(
        _tc_body,
        out_shape=jax.ShapeDtypeStruct((1, O), jnp.float32),
        in_specs=[
            pl.BlockSpec(memory_space=pltpu.SMEM),
            pl.BlockSpec(memory_space=pltpu.VMEM),
            pl.BlockSpec(memory_space=pltpu.VMEM),
            pl.BlockSpec(memory_space=pltpu.VMEM),
            pl.BlockSpec(memory_space=pltpu.VMEM),
            pl.BlockSpec(memory_space=pltpu.VMEM),
            pl.BlockSpec(memory_space=pltpu.VMEM),
            pl.BlockSpec(memory_space=pltpu.VMEM),
        ],
        out_specs=pl.BlockSpec(memory_space=pltpu.VMEM),
    )(idxv, degp, cntp, x, W1, Wfc, bias2, bfc2)


def kernel(x, edge_index, edge_weight, W1, bias, Wfc, bfc):
    ei = jnp.asarray(edge_index, jnp.int32)
    degp, cntp, idxv = _sc_hist(ei)
    y = _tc_finish(idxv, degp, cntp, x, W1, Wfc,
                   bias.reshape(1, H), bfc.reshape(1, O))
    return y.reshape(O)


# speedup vs baseline: 1.0125x; 1.0125x over previous
import jax
import jax.numpy as jnp
from jax import lax
from jax.experimental import pallas as pl
from jax.experimental.pallas import tpu as pltpu
from jax.experimental.pallas import tpu_sc as plsc

N = 10000
E = 160000
D = 256
H = 256
O = 256

NC = 2
NS = 16
L = 16
NW = NC * NS

CH = 9984
CH_LAST = E - 15 * CH
HH = CH // 2
HH_LAST = CH_LAST // 2
MIN_VREGS = CH_LAST // L
H_ITERS = HH_LAST // L
U = 8
ZU = 5
MU = 8

_INT32_MAX = 2147483647


def _sc_body(ei_hbm, deg_out, cnt_out, idx_out,
             ei_v, deg_v, cnt_v, stage_v, shared_mins, allmin_v, dma_sem,
             out_sem):
    c_idx = lax.axis_index("c")
    s_idx = lax.axis_index("s")
    wid = s_idx * NC + c_idx

    iota = lax.iota(jnp.int32, L)
    ones = jnp.full((L,), 1.0, jnp.float32)
    zeros = jnp.zeros((L,), jnp.float32)

    ei_dma = pltpu.async_copy(ei_hbm.at[:, pl.ds(s_idx * CH, CH_LAST)],
                              ei_v, dma_sem)

    def zero_body(i, carry):
        base = i * (L * ZU)
        for u in range(ZU):
            off = base + u * L
            deg_v[pl.ds(off, L)] = zeros
            cnt_v[pl.ds(off, L)] = zeros
        return carry
    lax.fori_loop(0, (N // L) // ZU, zero_body, 0)

    ei_dma.wait()

    def min_body(i, acc):
        base = i * (L * MU)
        for u in range(MU):
            off = base + u * L
            acc = jnp.minimum(acc, ei_v[0, pl.ds(off, L)])
        return acc
    acc = lax.fori_loop(0, MIN_VREGS // MU, min_body,
                        jnp.full((L,), _INT32_MAX, jnp.int32))

    hlen = jnp.where(s_idx == NS - 1, HH_LAST, HH)
    hbase = c_idx * hlen

    def deg_body(i, carry):
        base = i * (L * U)
        for u in range(U):
            off = base + u * L
            cvals = ei_v[1, pl.ds(hbase + off, L)]
            plsc.addupdate_scatter(deg_v, [cvals], ones,
                                   mask=(off + iota) < hlen)
        return carry
    lax.fori_loop(0, H_ITERS // U, deg_body, 0)

    deg_dma = pltpu.async_copy(deg_v, deg_out.at[wid], out_sem)

    stage_v[...] = acc
    pltpu.sync_copy(stage_v, shared_mins.at[pl.ds(s_idx * L, L)])
    plsc.subcore_barrier()
    pltpu.sync_copy(shared_mins, allmin_v)
    macc = allmin_v[pl.ds(0, L)]
    for t in range(1, NS):
        macc = jnp.minimum(macc, allmin_v[pl.ds(t * L, L)])
    gmin = jnp.min(macc)

    def cnt_body(i, carry):
        base = i * (L * U)
        for u in range(U):
            off = base + u * L
            cvals = ei_v[1, pl.ds(hbase + off, L)]
            rvals = ei_v[0, pl.ds(hbase + off, L)]
            plsc.addupdate_scatter(
                cnt_v, [rvals], ones,
                mask=((off + iota) < hlen) & (cvals == gmin))
        return carry
    lax.fori_loop(0, H_ITERS // U, cnt_body, 0)

    pltpu.sync_copy(cnt_v, cnt_out.at[wid])
    deg_dma.wait()

    @pl.when(wid == 0)
    def _():
        stage_v[...] = jnp.full((L,), 0, jnp.int32) + gmin
        pltpu.sync_copy(stage_v, idx_out)


@jax.jit
def _sc_hist(edge_index):
    kern = pl.kernel(
        _sc_body,
        out_type=(
            jax.ShapeDtypeStruct((NW, N), jnp.float32),
            jax.ShapeDtypeStruct((NW, N), jnp.float32),
            jax.ShapeDtypeStruct((L,), jnp.int32),
        ),
        mesh=plsc.VectorSubcoreMesh(core_axis_name="c", subcore_axis_name="s"),
        compiler_params=pltpu.CompilerParams(needs_layout_passes=False),
        scratch_types=[
            pltpu.VMEM((2, CH_LAST), jnp.int32),
            pltpu.VMEM((N,), jnp.float32),
            pltpu.VMEM((N,), jnp.float32),
            pltpu.VMEM((L,), jnp.int32),
            pltpu.VMEM_SHARED((NS * L,), jnp.int32),
            pltpu.VMEM((NS * L,), jnp.int32),
            pltpu.SemaphoreType.DMA,
            pltpu.SemaphoreType.DMA,
        ],
    )
    return kern(edge_index)


def _tc_body(idx_ref, degp_ref, cntp_ref, x_ref, w1_ref, wfc_ref,
             bias_ref, bfc_ref, y_ref):
    deg = jnp.sum(degp_ref[...], axis=0, keepdims=True)
    cnt = jnp.sum(cntp_ref[...], axis=0, keepdims=True)
    w = cnt * jnp.sqrt(deg)
    idx = idx_ref[0]
    onehot = (lax.broadcasted_iota(jnp.int32, (1, N), 1) == idx)
    scale = jnp.sqrt(jnp.sum(jnp.where(onehot, deg, 0.0)))
    s = lax.dot_general(w, x_ref[...], (((1,), (0,)), ((), ())),
                        preferred_element_type=jnp.float32,
                        precision=lax.Precision.HIGHEST)
    z = lax.dot_general(s, w1_ref[...], (((1,), (1,)), ((), ())),
                        preferred_element_type=jnp.float32,
                        precision=lax.Precision.HIGHEST)
    out_row = scale * z + bias_ref[...]
    y = lax.dot_general(out_row, wfc_ref[...], (((1,), (1,)), ((), ())),
                        preferred_element_type=jnp.float32,
                        precision=lax.Precision.HIGHEST) + bfc_ref[...]
    y_ref[...] = y


@jax.jit
def _tc_finish(idxv, degp, cntp, x, W1, Wfc, bias2, bfc2):
    return pl.pallas_call(
        _tc_body,
        out_shape=jax.ShapeDtypeStruct((1, O), jnp.float32),
        in_specs=[
            pl.BlockSpec(memory_space=pltpu.SMEM),
            pl.BlockSpec(memory_space=pltpu.VMEM),
            pl.BlockSpec(memory_space=pltpu.VMEM),
            pl.BlockSpec(memory_space=pltpu.VMEM),
            pl.BlockSpec(memory_space=pltpu.VMEM),
            pl.BlockSpec(memory_space=pltpu.VMEM),
            pl.BlockSpec(memory_space=pltpu.VMEM),
            pl.BlockSpec(memory_space=pltpu.VMEM),
        ],
        out_specs=pl.BlockSpec(memory_space=pltpu.VMEM),
    )(idxv, degp, cntp, x, W1, Wfc, bias2, bfc2)


def kernel(x, edge_index, edge_weight, W1, bias, Wfc, bfc):
    ei = jnp.asarray(edge_index, jnp.int32)
    degp, cntp, idxv = _sc_hist(ei)
    y = _tc_finish(idxv, degp, cntp, x, W1, Wfc,
                   bias.reshape(1, H), bfc.reshape(1, O))
    return y.reshape(O)
